# baseline (speedup 1.0000x reference)
import functools

import jax
import jax.numpy as jnp
from jax import lax
from jax.experimental import pallas as pl
from jax.experimental.pallas import tpu as pltpu
from jax.experimental.pallas import tpu_sc as plsc

B, K, N = 8, 4, 10
BATCHES = B * K
C, H, W = 384, 32, 32
NBOX = B * K * N
NW = 32
BOX_PER_W = NBOX // NW
IDX_PER_W = BOX_PER_W * C
CHW = C * H * W
ROWS_PER_BATCH = CHW // 128
ROWS_PER_CH = H * W // 128
CHUNK = 128
PASS_BOX = 2
NPASS = BOX_PER_W // PASS_BOX
PASS_IDX = PASS_BOX * C
PASS_CHUNKS = PASS_IDX // CHUNK


def _body(table_hbm, bboxes_hbm, out_hbm, bb_v, rb_v, off_v, idx_v,
          rows_v, out_v, sem):
    wid = lax.axis_index("s") * 2 + lax.axis_index("c")

    pltpu.sync_copy(bboxes_hbm, bb_v)

    lane = lax.broadcasted_iota(jnp.int32, (16,), 0)
    box = jnp.minimum(wid * BOX_PER_W + lane, NBOX - 1)

    def field(f):
        p = box * 4 + f
        return plsc.load_gather(bb_v, [p >> 4, p & 15])

    x0, y0, bw, bh = field(0), field(1), field(2), field(3)
    cx = (x0 + (bw >> 1)) >> 4
    cy = (y0 + (bh >> 1)) >> 4
    rb = wid * ROWS_PER_BATCH + (cy >> 2)
    col = ((cy & 3) << 5) | cx
    rb_v[pl.ds(0, 16)] = rb
    rb_v[pl.ds(16, 16)] = rb
    off_v[pl.ds(0, 16)] = col
    off_v[pl.ds(16, 16)] = col

    for b in range(BOX_PER_W):
        rb_b = plsc.load_gather(rb_v, [jnp.full((16,), 16 + b, jnp.int32)])
        for j in range(C // 16):
            idx_v[pl.ds(b * C + j * 16, 16)] = (
                rb_b + (lane + j * 16) * ROWS_PER_CH)

    for p in range(NPASS):
        for j in range(PASS_CHUNKS):
            pltpu.async_copy(
                table_hbm.at[idx_v.at[pl.ds(p * PASS_IDX + j * CHUNK, CHUNK)]],
                rows_v.at[pl.ds(j * CHUNK, CHUNK)], sem)
        pltpu.make_async_copy(
            table_hbm.at[pl.ds(0, PASS_IDX)], rows_v, sem).wait()

        for bl in range(PASS_BOX):
            b = p * PASS_BOX + bl
            ob = plsc.load_gather(off_v, [jnp.full((16,), 16 + b, jnp.int32)])
            for j in range(C // 16):
                out_v[pl.ds(b * C + j * 16, 16)] = plsc.load_gather(
                    rows_v, [bl * C + j * 16 + lane, ob])

    pltpu.sync_copy(out_v, out_hbm.at[pl.ds(wid * IDX_PER_W, IDX_PER_W)])


@jax.jit
def _center_pool(input, bboxes):
    mesh = plsc.VectorSubcoreMesh(core_axis_name="c", subcore_axis_name="s")
    run = functools.partial(
        pl.kernel,
        mesh=mesh,
        out_type=jax.ShapeDtypeStruct((NBOX * C,), jnp.float32),
        scratch_types=[
            pltpu.VMEM((NBOX * 4 // 16, 16), jnp.int32),
            pltpu.VMEM((32,), jnp.int32),
            pltpu.VMEM((32,), jnp.int32),
            pltpu.VMEM((IDX_PER_W,), jnp.int32),
            pltpu.VMEM((PASS_IDX, 128), jnp.float32),
            pltpu.VMEM((IDX_PER_W,), jnp.float32),
            pltpu.SemaphoreType.DMA,
        ],
        compiler_params=pltpu.CompilerParams(
            needs_layout_passes=False, use_tc_tiling_on_sc=True
        ),
    )(_body)
    out = run(input.reshape(BATCHES * CHW // 128, 128),
              bboxes.reshape(NBOX * 4 // 16, 16))
    return out.reshape(B, K * N, C)


def kernel(input, bboxes):
    return _center_pool(input, bboxes)


# speedup vs baseline: 1.0946x; 1.0946x over previous
import functools

import jax
import jax.numpy as jnp
from jax import lax
from jax.experimental import pallas as pl
from jax.experimental.pallas import tpu as pltpu
from jax.experimental.pallas import tpu_sc as plsc

B, K, N = 8, 4, 10
BATCHES = B * K
C, H, W = 384, 32, 32
NBOX = B * K * N
NW = 32
BOX_PER_W = NBOX // NW
IDX_PER_W = BOX_PER_W * C


def _body(input_hbm, bboxes_hbm, out_hbm, bb_v, off_v, rows_v,
          out_v, sem):
    wid = lax.axis_index("s") * 2 + lax.axis_index("c")

    pltpu.sync_copy(bboxes_hbm, bb_v)

    lane = lax.broadcasted_iota(jnp.int32, (16,), 0)
    box = jnp.minimum(wid * BOX_PER_W + lane, NBOX - 1)

    def field(f):
        p = box * 4 + f
        return plsc.load_gather(bb_v, [p >> 4, p & 15])

    x0, y0, bw, bh = field(0), field(1), field(2), field(3)
    cx = (x0 + (bw >> 1)) >> 4
    cy = (y0 + (bh >> 1)) >> 4
    off_v[pl.ds(0, 16)] = cx
    off_v[pl.ds(16, 16)] = cx

    for b in range(BOX_PER_W):
        cy_b = cy[b]
        pltpu.async_copy(
            input_hbm.at[wid, :, cy_b, :],
            rows_v.at[pl.ds(b * C, C)], sem)
    pltpu.make_async_copy(
        input_hbm.at[0, :, 0, :], rows_v.at[pl.ds(0, C)], sem,
    ).wait()
    for _ in range(BOX_PER_W - 1):
        pltpu.make_async_copy(
            input_hbm.at[0, :, 0, :], rows_v.at[pl.ds(0, C)], sem,
        ).wait()

    for b in range(BOX_PER_W):
        ob = plsc.load_gather(off_v, [jnp.full((16,), 16 + b, jnp.int32)])
        for j in range(C // 16):
            out_v[pl.ds(b * C + j * 16, 16)] = plsc.load_gather(
                rows_v, [b * C + j * 16 + lane, ob])

    pltpu.sync_copy(out_v, out_hbm.at[pl.ds(wid * IDX_PER_W, IDX_PER_W)])


@jax.jit
def _center_pool(input, bboxes):
    mesh = plsc.VectorSubcoreMesh(core_axis_name="c", subcore_axis_name="s")
    run = functools.partial(
        pl.kernel,
        mesh=mesh,
        out_type=jax.ShapeDtypeStruct((NBOX * C,), jnp.float32),
        scratch_types=[
            pltpu.VMEM((NBOX * 4 // 16, 16), jnp.int32),
            pltpu.VMEM((32,), jnp.int32),
            pltpu.VMEM((IDX_PER_W, W), jnp.float32),
            pltpu.VMEM((IDX_PER_W,), jnp.float32),
            pltpu.SemaphoreType.DMA,
        ],
        compiler_params=pltpu.CompilerParams(
            needs_layout_passes=False, use_tc_tiling_on_sc=False
        ),
    )(_body)
    out = run(input, bboxes.reshape(NBOX * 4 // 16, 16))
    return out.reshape(B, K * N, C)


def kernel(input, bboxes):
    return _center_pool(input, bboxes)


# speedup vs baseline: 9.8278x; 8.9782x over previous
import functools

import jax
import jax.numpy as jnp
from jax import lax
from jax.experimental import pallas as pl
from jax.experimental.pallas import tpu as pltpu
from jax.experimental.pallas import tpu_sc as plsc

B, K, N = 8, 4, 10
BATCHES = B * K
C, H, W = 384, 32, 32
NBOX = B * K * N
NW = 32
BOX_PER_W = NBOX // NW
HW = H * W


def _body(input_hbm, bboxes_hbm, out_hbm, bb_v, idx_v, rows_v, sem):
    table_hbm = input_hbm.reshape(BATCHES * HW, C)
    wid = lax.axis_index("s") * 2 + lax.axis_index("c")

    pltpu.sync_copy(bboxes_hbm, bb_v)

    lane = lax.broadcasted_iota(jnp.int32, (16,), 0)
    box = jnp.minimum(wid * BOX_PER_W + lane, NBOX - 1)

    def field(f):
        p = box * 4 + f
        return plsc.load_gather(bb_v, [p >> 4, p & 15])

    x0, y0, bw, bh = field(0), field(1), field(2), field(3)
    cx = (x0 + (bw >> 1)) >> 4
    cy = (y0 + (bh >> 1)) >> 4
    idx_v[pl.ds(0, 16)] = wid * HW + cy * W + cx

    pltpu.async_copy(
        table_hbm.at[idx_v.at[pl.ds(0, 16)]], rows_v, sem)
    pltpu.make_async_copy(
        table_hbm.at[pl.ds(0, 16)], rows_v, sem).wait()

    pltpu.sync_copy(rows_v, out_hbm.at[wid])


@jax.jit
def _center_pool(input, bboxes):
    mesh = plsc.VectorSubcoreMesh(core_axis_name="c", subcore_axis_name="s")
    run = functools.partial(
        pl.kernel,
        mesh=mesh,
        out_type=jax.ShapeDtypeStruct((NW, 16, C), jnp.float32),
        scratch_types=[
            pltpu.VMEM((NBOX * 4 // 16, 16), jnp.int32),
            pltpu.VMEM((16,), jnp.int32),
            pltpu.VMEM((16, C), jnp.float32),
            pltpu.SemaphoreType.DMA,
        ],
        compiler_params=pltpu.CompilerParams(
            needs_layout_passes=False, use_tc_tiling_on_sc=True
        ),
    )(_body)
    input_t = jnp.transpose(input, (0, 2, 3, 1))
    out = run(input_t, bboxes.reshape(NBOX * 4 // 16, 16))
    return out[:, :BOX_PER_W, :].reshape(B, K * N, C)


def kernel(input, bboxes):
    return _center_pool(input, bboxes)
